# baseline (speedup 1.0000x reference)
import jax
import jax.numpy as jnp
from jax import lax
from jax.experimental import pallas as pl
from jax.experimental.pallas import tpu as pltpu
from jax.experimental.pallas import tpu_sc as plsc

_B, _S = 16384, 200
_D = 16
_NC, _NS, _L = 2, 16, 16
_NW = _NC * _NS
_IW = _B // _NW
_NG = _IW // _L
_NBLK = _S // 8

_GDN = lax.GatherDimensionNumbers(
    offset_dims=(), collapsed_slice_dims=(0,), start_index_map=(0,))


def _lane_lookup(col, iv):
    return lax.gather(col, iv[:, None], _GDN, (1,),
                      mode=lax.GatherScatterMode.PROMISE_IN_BOUNDS)


def _body(tl_idx, ag_idx, tabcols, tl_out, ag_out,
          tab_v, idx_v, out_v, sin0, sin1, sout0, sout1):
    wid = lax.axis_index("s") * _NC + lax.axis_index("c")
    i0 = wid * _IW
    pltpu.sync_copy(tabcols, tab_v)
    sin = (sin0, sin1)
    sout = (sout0, sout1)

    for f, (idx_hbm, out_hbm) in enumerate(((tl_idx, tl_out),
                                            (ag_idx, ag_out))):
        cols = [tab_v[f * _D + d, pl.ds(0, _L)] for d in range(_D)]

        def start_in(b, q, idx_hbm=idx_hbm):
            pltpu.async_copy(idx_hbm.at[pl.ds(b * 8, 8), pl.ds(i0, _IW)],
                             idx_v.at[q], sin[q])

        def wait_in(q, idx_hbm=idx_hbm):
            pltpu.make_async_copy(
                idx_hbm.at[pl.ds(0, 8), pl.ds(i0, _IW)],
                idx_v.at[q], sin[q]).wait()

        def wait_out(p, out_hbm=out_hbm):
            pltpu.make_async_copy(
                out_v.at[p], out_hbm.at[0, :, pl.ds(i0, _IW)],
                sout[p]).wait()

        def do_block(b, q, cols=cols, out_hbm=out_hbm):
            wait_in(q)

            def h_body(h, _):
                for p in (0, 1):
                    jl = 2 * h + p
                    not_first = jnp.logical_not(
                        jnp.logical_and(b == 0, h == 0))

                    @pl.when(not_first)
                    def _(p=p):
                        wait_out(p)

                    for g in range(_NG):
                        iv = idx_v[q, jl, pl.ds(g * _L, _L)]
                        for d in range(_D):
                            out_v[p, d, pl.ds(g * _L, _L)] = _lane_lookup(
                                cols[d], iv)
                    pltpu.async_copy(
                        out_v.at[p],
                        out_hbm.at[b * 8 + jl, :, pl.ds(i0, _IW)],
                        sout[p])
                return 0

            lax.fori_loop(0, 4, h_body, 0)

            @pl.when(b + 2 < _NBLK)
            def _():
                start_in(b + 2, q)

        start_in(0, 0)
        start_in(1, 1)

        def k_body(k, _):
            do_block(2 * k, 0)
            do_block(2 * k + 1, 1)
            return 0

        lax.fori_loop(0, (_NBLK - 1) // 2, k_body, 0)
        do_block(jnp.int32(_NBLK - 1), 0)
        wait_out(0)
        wait_out(1)


@jax.jit
def _run(tl_idx_t, ag_idx_t, tabcols):
    mesh = plsc.VectorSubcoreMesh(core_axis_name="c", subcore_axis_name="s",
                                  num_cores=_NC, num_subcores=_NS)
    fn = pl.kernel(
        _body,
        out_type=(
            jax.ShapeDtypeStruct((_S, _D, _B), jnp.float32),
            jax.ShapeDtypeStruct((_S, _D, _B), jnp.float32),
        ),
        mesh=mesh,
        scratch_types=[
            pltpu.VMEM((2 * _D, 128), jnp.float32),
            pltpu.VMEM((2, 8, _IW), jnp.int32),
            pltpu.VMEM((2, _D, _IW), jnp.float32),
            pltpu.SemaphoreType.DMA,
            pltpu.SemaphoreType.DMA,
            pltpu.SemaphoreType.DMA,
            pltpu.SemaphoreType.DMA,
        ],
        compiler_params=pltpu.CompilerParams(
            needs_layout_passes=False,
            use_tc_tiling_on_sc=True,
        ),
    )
    return fn(tl_idx_t, ag_idx_t, tabcols)


def kernel(traffic_light_state, agent_type, tl_table, agent_table):
    tabcols = jnp.zeros((2 * _D, 128), jnp.float32)
    tabcols = tabcols.at[:_D, :9].set(tl_table.T)
    tabcols = tabcols.at[_D:, :5].set(agent_table.T)
    tl_p, ag_p = _run(traffic_light_state.T, agent_type.T, tabcols)
    return (jnp.transpose(tl_p, (2, 0, 1)), jnp.transpose(ag_p, (2, 0, 1)))


# speedup vs baseline: 1.4576x; 1.4576x over previous
import jax
import jax.numpy as jnp
from jax import lax
from jax.experimental import pallas as pl
from jax.experimental.pallas import tpu as pltpu
from jax.experimental.pallas import tpu_sc as plsc

_B, _S = 16384, 200
_D = 16
_NC, _NS, _L = 2, 16, 16
_NW = _NC * _NS
_IW = _B // _NW
_NG = _IW // _L
_NBLK = _S // 8

_GDN = lax.GatherDimensionNumbers(
    offset_dims=(), collapsed_slice_dims=(0,), start_index_map=(0,))


def _lane_lookup(col, iv):
    return lax.gather(col, iv[:, None], _GDN, (1,),
                      mode=lax.GatherScatterMode.PROMISE_IN_BOUNDS)


def _body(tl_idx, ag_idx, tabcols, tl_out, ag_out,
          tab_v, idx_v, out_v, sin0, sin1, sout0, sout1):
    wid = lax.axis_index("s") * _NC + lax.axis_index("c")
    i0 = wid * _IW
    pltpu.sync_copy(tabcols, tab_v)
    sin = (sin0, sin1)
    sout = (sout0, sout1)

    for f, (idx_hbm, out_hbm) in enumerate(((tl_idx, tl_out),
                                            (ag_idx, ag_out))):
        cols = [tab_v[f * _D + d, pl.ds(0, _L)] for d in range(_D)]

        def start_in(b, q, idx_hbm=idx_hbm):
            pltpu.async_copy(idx_hbm.at[pl.ds(b * 8, 8), pl.ds(i0, _IW)],
                             idx_v.at[q], sin[q])

        def wait_in(q, idx_hbm=idx_hbm):
            pltpu.make_async_copy(
                idx_hbm.at[pl.ds(0, 8), pl.ds(i0, _IW)],
                idx_v.at[q], sin[q]).wait()

        def wait_out(p, out_hbm=out_hbm):
            pltpu.make_async_copy(
                out_v.at[p], out_hbm.at[0, :, pl.ds(i0, _IW)],
                sout[p]).wait()

        def do_block(b, q, cols=cols, out_hbm=out_hbm):
            wait_in(q)

            def h_body(h, _):
                for p in (0, 1):
                    jl = 2 * h + p
                    not_first = jnp.logical_not(
                        jnp.logical_and(b == 0, h == 0))

                    @pl.when(not_first)
                    def _(p=p):
                        wait_out(p)

                    def g_body(g, _, p=p, q=q, jl=jl):
                        for u in range(2):
                            gl = 2 * g + u
                            iv = idx_v[q, jl, pl.ds(gl * _L, _L)]
                            for d in range(_D):
                                out_v[p, d, pl.ds(gl * _L, _L)] = (
                                    _lane_lookup(cols[d], iv))
                        return 0

                    lax.fori_loop(0, _NG // 2, g_body, 0)
                    pltpu.async_copy(
                        out_v.at[p],
                        out_hbm.at[b * 8 + jl, :, pl.ds(i0, _IW)],
                        sout[p])
                return 0

            lax.fori_loop(0, 4, h_body, 0)

            @pl.when(b + 2 < _NBLK)
            def _():
                start_in(b + 2, q)

        start_in(0, 0)
        start_in(1, 1)

        def k_body(k, _):
            do_block(2 * k, 0)
            do_block(2 * k + 1, 1)
            return 0

        lax.fori_loop(0, (_NBLK - 1) // 2, k_body, 0)
        do_block(jnp.int32(_NBLK - 1), 0)
        wait_out(0)
        wait_out(1)


@jax.jit
def _run(tl_idx_t, ag_idx_t, tabcols):
    mesh = plsc.VectorSubcoreMesh(core_axis_name="c", subcore_axis_name="s",
                                  num_cores=_NC, num_subcores=_NS)
    fn = pl.kernel(
        _body,
        out_type=(
            jax.ShapeDtypeStruct((_S, _D, _B), jnp.float32),
            jax.ShapeDtypeStruct((_S, _D, _B), jnp.float32),
        ),
        mesh=mesh,
        scratch_types=[
            pltpu.VMEM((2 * _D, 128), jnp.float32),
            pltpu.VMEM((2, 8, _IW), jnp.int32),
            pltpu.VMEM((2, _D, _IW), jnp.float32),
            pltpu.SemaphoreType.DMA,
            pltpu.SemaphoreType.DMA,
            pltpu.SemaphoreType.DMA,
            pltpu.SemaphoreType.DMA,
        ],
        compiler_params=pltpu.CompilerParams(
            needs_layout_passes=False,
            use_tc_tiling_on_sc=True,
        ),
    )
    return fn(tl_idx_t, ag_idx_t, tabcols)


def kernel(traffic_light_state, agent_type, tl_table, agent_table):
    tabcols = jnp.zeros((2 * _D, 128), jnp.float32)
    tabcols = tabcols.at[:_D, :9].set(tl_table.T)
    tabcols = tabcols.at[_D:, :5].set(agent_table.T)
    tl_p, ag_p = _run(traffic_light_state.T, agent_type.T, tabcols)
    return (jnp.transpose(tl_p, (2, 0, 1)), jnp.transpose(ag_p, (2, 0, 1)))
